# baseline (speedup 1.0000x reference)
import functools

import jax
import jax.numpy as jnp
from jax import lax
from jax.experimental import pallas as pl
from jax.experimental.pallas import tpu as pltpu
from jax.experimental.pallas import tpu_sc as plsc

_BATCH = 4096
_IN_DIM = 8192
_OUT_DIM = 16384

_NW = 32
_ROWS_PER_W = _BATCH // _NW
_R = 4
_NBLK = _ROWS_PER_W // _R
_OC = 1024
_NCH = _OUT_DIM // _OC
_GPC = _OC // 16

_mesh = plsc.VectorSubcoreMesh(core_axis_name="c", subcore_axis_name="s")


@functools.partial(
    pl.kernel,
    out_type=jax.ShapeDtypeStruct((_BATCH * _OUT_DIM,), jnp.float32),
    mesh=_mesh,
    compiler_params=pltpu.CompilerParams(needs_layout_passes=False),
    scratch_types=[
        pltpu.VMEM((_OUT_DIM,), jnp.int32),
        pltpu.VMEM((_OUT_DIM,), jnp.int32),
        pltpu.VMEM((_OUT_DIM,), jnp.float32),
        pltpu.VMEM((_OUT_DIM,), jnp.float32),
        pltpu.VMEM((_OUT_DIM,), jnp.float32),
        pltpu.VMEM((_R * _IN_DIM,), jnp.float32),
        pltpu.VMEM((_R * _OC,), jnp.float32),
    ],
)
def _logic_fwd(x_hbm, ia_hbm, ib_hbm, ca_hbm, cb_hbm, cab_hbm, out_hbm,
               ia_v, ib_v, ca_v, cb_v, cab_v, rows_v, ob_v):
    wid = lax.axis_index("s") * 2 + lax.axis_index("c")
    pltpu.sync_copy(ia_hbm, ia_v)
    pltpu.sync_copy(ib_hbm, ib_v)
    pltpu.sync_copy(ca_hbm, ca_v)
    pltpu.sync_copy(cb_hbm, cb_v)
    pltpu.sync_copy(cab_hbm, cab_v)
    row0 = wid * _ROWS_PER_W

    def blk_body(blk, _):
        rbase = row0 + blk * _R
        pltpu.sync_copy(x_hbm.at[pl.ds(rbase * _IN_DIM, _R * _IN_DIM)],
                        rows_v)

        def ch_body(ch, _):
            def g_body(g, _):
                gbase = ch * _OC + g * 16
                ia = ia_v[pl.ds(gbase, 16)]
                ib = ib_v[pl.ds(gbase, 16)]
                ca = ca_v[pl.ds(gbase, 16)]
                cb = cb_v[pl.ds(gbase, 16)]
                cab = cab_v[pl.ds(gbase, 16)]
                for r in range(_R):
                    a = plsc.load_gather(rows_v, [ia + (r * _IN_DIM)])
                    b = plsc.load_gather(rows_v, [ib + (r * _IN_DIM)])
                    ob_v[pl.ds(r * _OC + g * 16, 16)] = (
                        ca * a + cb * b + cab * (a * b))
                return 0

            lax.fori_loop(0, _GPC, g_body, 0)
            for r in range(_R):
                pltpu.sync_copy(
                    ob_v.at[pl.ds(r * _OC, _OC)],
                    out_hbm.at[pl.ds((rbase + r) * _OUT_DIM + ch * _OC, _OC)])
            return 0

        lax.fori_loop(0, _NCH, ch_body, 0)
        return 0

    lax.fori_loop(0, _NBLK, blk_body, 0)


@jax.jit
def kernel(x, weights, idx_a, idx_b):
    w = jax.nn.softmax(weights, axis=-1)
    ca = w[:, 2] + w[:, 3]
    cb = w[:, 4] + w[:, 5]
    cab = w[:, 1] - w[:, 2] - w[:, 4]
    out = _logic_fwd(x.reshape(-1), idx_a, idx_b, ca, cb, cab)
    return out.reshape(_BATCH, _OUT_DIM)


# speedup vs baseline: 1.0974x; 1.0974x over previous
import functools

import jax
import jax.numpy as jnp
from jax import lax
from jax.experimental import pallas as pl
from jax.experimental.pallas import tpu as pltpu
from jax.experimental.pallas import tpu_sc as plsc

_BATCH = 4096
_IN_DIM = 8192
_OUT_DIM = 16384

_NW = 32
_ROWS_PER_W = _BATCH // _NW
_R = 4
_NBLK = _ROWS_PER_W // _R
_OC = 1024
_NCH = _OUT_DIM // _OC
_GPC = _OC // 16

_mesh = plsc.VectorSubcoreMesh(core_axis_name="c", subcore_axis_name="s")


@functools.partial(
    pl.kernel,
    out_type=jax.ShapeDtypeStruct((_BATCH * _OUT_DIM,), jnp.float32),
    mesh=_mesh,
    compiler_params=pltpu.CompilerParams(needs_layout_passes=False),
    scratch_types=[
        pltpu.VMEM((_OUT_DIM,), jnp.int32),
        pltpu.VMEM((_OUT_DIM,), jnp.int32),
        pltpu.VMEM((_OUT_DIM,), jnp.float32),
        pltpu.VMEM((2 * _R * _IN_DIM,), jnp.float32),
        pltpu.VMEM((2 * _R * _OC,), jnp.float32),
        pltpu.SemaphoreType.DMA,
        pltpu.SemaphoreType.DMA,
    ],
)
def _logic_fwd(x_hbm, iaib_hbm, cacb_hbm, cab_hbm, out_hbm,
               iaib_v, cacb_v, cab_v, rows_v, ob_v, row_sem, out_sem):
    wid = lax.axis_index("s") * 2 + lax.axis_index("c")
    pltpu.sync_copy(iaib_hbm, iaib_v)
    pltpu.sync_copy(cacb_hbm, cacb_v)
    pltpu.sync_copy(cab_hbm, cab_v)
    row0 = wid * _ROWS_PER_W

    def row_load(blk, buf):
        pltpu.async_copy(
            x_hbm.at[pl.ds((row0 + blk * _R) * _IN_DIM, _R * _IN_DIM)],
            rows_v.at[pl.ds(buf * (_R * _IN_DIM), _R * _IN_DIM)], row_sem)

    def out_store(blk, ch, buf):
        rbase = row0 + blk * _R
        for r in range(_R):
            pltpu.async_copy(
                ob_v.at[pl.ds(buf * (_R * _OC) + r * _OC, _OC)],
                out_hbm.at[pl.ds((rbase + r) * _OUT_DIM + ch * _OC, _OC)],
                out_sem)

    def out_drain():
        for r in range(_R):
            pltpu.make_async_copy(
                ob_v.at[pl.ds(r * _OC, _OC)],
                out_hbm.at[pl.ds(r * _OUT_DIM, _OC)], out_sem).wait()

    row_load(0, 0)
    row_load(1, 1)

    def blk_body(blk, _):
        rbuf = lax.rem(blk, 2)
        pltpu.make_async_copy(
            x_hbm.at[pl.ds(0, _R * _IN_DIM)],
            rows_v.at[pl.ds(0, _R * _IN_DIM)], row_sem).wait()

        def ch_body(ch, _):
            obuf = lax.rem(ch, 2)
            lin = blk * _NCH + ch

            @pl.when(lin >= 2)
            def _():
                out_drain()

            def g_body(g, _):
                gbase = ch * _OC + g * 16
                v = iaib_v[pl.ds(gbase, 16)]
                ia = lax.bitwise_and(v, 0xFFFF)
                ib = lax.shift_right_logical(v, 16)
                cc = cacb_v[pl.ds(gbase, 16)]
                ca = plsc.bitcast(lax.shift_left(cc, 16), jnp.float32)
                cb = plsc.bitcast(
                    lax.bitwise_and(cc, jnp.int32(-65536)), jnp.float32)
                cab = cab_v[pl.ds(gbase, 16)]
                for r in range(_R):
                    rowslc = rows_v.at[
                        pl.ds(rbuf * (_R * _IN_DIM) + r * _IN_DIM, _IN_DIM)]
                    a = plsc.load_gather(rowslc, [ia])
                    b = plsc.load_gather(rowslc, [ib])
                    ob_v[pl.ds(obuf * (_R * _OC) + r * _OC + g * 16, 16)] = (
                        a * (ca + cab * b) + cb * b)
                return 0

            lax.fori_loop(0, _GPC, g_body, 0)
            out_store(blk, ch, obuf)
            return 0

        lax.fori_loop(0, _NCH, ch_body, 0)

        @pl.when(blk + 2 < _NBLK)
        def _():
            row_load(blk + 2, rbuf)
        return 0

    lax.fori_loop(0, _NBLK, blk_body, 0)
    out_drain()
    out_drain()


@jax.jit
def kernel(x, weights, idx_a, idx_b):
    w = jax.nn.softmax(weights, axis=-1)
    ca = w[:, 2] + w[:, 3]
    cb = w[:, 4] + w[:, 5]
    cab = w[:, 1] - w[:, 2] - w[:, 4]
    iaib = idx_a | (idx_b << 16)
    ca16 = jax.lax.bitcast_convert_type(
        ca.astype(jnp.bfloat16), jnp.uint16).astype(jnp.int32)
    cb16 = jax.lax.bitcast_convert_type(
        cb.astype(jnp.bfloat16), jnp.uint16).astype(jnp.int32)
    cacb = (cb16 << 16) | ca16
    out = _logic_fwd(x.reshape(-1), iaib, cacb, cab)
    return out.reshape(_BATCH, _OUT_DIM)


# speedup vs baseline: 2.8443x; 2.5918x over previous
import functools

import jax
import jax.numpy as jnp
from jax import lax
from jax.experimental import pallas as pl
from jax.experimental.pallas import tpu as pltpu
from jax.experimental.pallas import tpu_sc as plsc

_BATCH = 4096
_IN_DIM = 8192
_OUT_DIM = 16384

_NW = 32
_ROWS_PER_W = _BATCH // _NW
_R = 4
_NBLK = _ROWS_PER_W // _R
_OC = 1024
_NCH = _OUT_DIM // _OC
_GPC = _OC // 16

_mesh = plsc.VectorSubcoreMesh(core_axis_name="c", subcore_axis_name="s")


@functools.partial(
    pl.kernel,
    out_type=jax.ShapeDtypeStruct((_BATCH * _OUT_DIM,), jnp.float32),
    mesh=_mesh,
    compiler_params=pltpu.CompilerParams(needs_layout_passes=False),
    scratch_types=[
        pltpu.VMEM((_OUT_DIM,), jnp.int32),
        pltpu.VMEM((_OUT_DIM,), jnp.int32),
        pltpu.VMEM((_OUT_DIM,), jnp.float32),
        pltpu.VMEM((2 * _R * _IN_DIM,), jnp.float32),
        pltpu.VMEM((2 * _R * _OC,), jnp.float32),
        pltpu.SemaphoreType.DMA,
        pltpu.SemaphoreType.DMA,
    ],
)
def _logic_fwd(x_hbm, iaib_hbm, cacb_hbm, cab_hbm, out_hbm,
               iaib_v, cacb_v, cab_v, rows_v, ob_v, row_sem, out_sem):
    wid = lax.axis_index("s") * 2 + lax.axis_index("c")
    pltpu.sync_copy(iaib_hbm, iaib_v)
    pltpu.sync_copy(cacb_hbm, cacb_v)
    pltpu.sync_copy(cab_hbm, cab_v)
    row0 = wid * _ROWS_PER_W

    def row_load(blk, buf):
        pltpu.async_copy(
            x_hbm.at[pl.ds((row0 + blk * _R) * _IN_DIM, _R * _IN_DIM)],
            rows_v.at[pl.ds(buf * (_R * _IN_DIM), _R * _IN_DIM)], row_sem)

    def out_store(blk, ch, buf):
        rbase = row0 + blk * _R
        for r in range(_R):
            pltpu.async_copy(
                ob_v.at[pl.ds(buf * (_R * _OC) + r * _OC, _OC)],
                out_hbm.at[pl.ds((rbase + r) * _OUT_DIM + ch * _OC, _OC)],
                out_sem)

    def out_drain():
        for r in range(_R):
            pltpu.make_async_copy(
                ob_v.at[pl.ds(r * _OC, _OC)],
                out_hbm.at[pl.ds(r * _OUT_DIM, _OC)], out_sem).wait()

    row_load(0, 0)
    row_load(1, 1)

    def blk_body(blk, _):
        rbuf = lax.rem(blk, 2)
        pltpu.make_async_copy(
            x_hbm.at[pl.ds(0, _R * _IN_DIM)],
            rows_v.at[pl.ds(0, _R * _IN_DIM)], row_sem).wait()

        def ch_body(ch, _):
            obuf = lax.rem(ch, 2)
            lin = blk * _NCH + ch

            @pl.when(lin >= 2)
            def _():
                out_drain()

            @plsc.parallel_loop(0, _GPC, 1, unroll=4)
            def g_body(g):
                gbase = ch * _OC + g * 16
                v = iaib_v[pl.ds(gbase, 16)]
                ia = lax.bitwise_and(v, 0xFFFF)
                ib = lax.shift_right_logical(v, 16)
                cc = cacb_v[pl.ds(gbase, 16)]
                ca = plsc.bitcast(lax.shift_left(cc, 16), jnp.float32)
                cb = plsc.bitcast(
                    lax.bitwise_and(cc, jnp.int32(-65536)), jnp.float32)
                cab = cab_v[pl.ds(gbase, 16)]
                for r in range(_R):
                    rowslc = rows_v.at[
                        pl.ds(rbuf * (_R * _IN_DIM) + r * _IN_DIM, _IN_DIM)]
                    a = plsc.load_gather(rowslc, [ia])
                    b = plsc.load_gather(rowslc, [ib])
                    ob_v[pl.ds(obuf * (_R * _OC) + r * _OC + g * 16, 16)] = (
                        a * (ca + cab * b) + cb * b)

            out_store(blk, ch, obuf)
            return 0

        lax.fori_loop(0, _NCH, ch_body, 0)

        @pl.when(blk + 2 < _NBLK)
        def _():
            row_load(blk + 2, rbuf)
        return 0

    lax.fori_loop(0, _NBLK, blk_body, 0)
    out_drain()
    out_drain()


@jax.jit
def kernel(x, weights, idx_a, idx_b):
    w = jax.nn.softmax(weights, axis=-1)
    ca = w[:, 2] + w[:, 3]
    cb = w[:, 4] + w[:, 5]
    cab = w[:, 1] - w[:, 2] - w[:, 4]
    iaib = idx_a | (idx_b << 16)
    ca16 = jax.lax.bitcast_convert_type(
        ca.astype(jnp.bfloat16), jnp.uint16).astype(jnp.int32)
    cb16 = jax.lax.bitcast_convert_type(
        cb.astype(jnp.bfloat16), jnp.uint16).astype(jnp.int32)
    cacb = (cb16 << 16) | ca16
    out = _logic_fwd(x.reshape(-1), iaib, cacb, cab)
    return out.reshape(_BATCH, _OUT_DIM)


# speedup vs baseline: 4.4531x; 1.5656x over previous
import functools

import jax
import jax.numpy as jnp
from jax import lax
from jax.experimental import pallas as pl
from jax.experimental.pallas import tpu as pltpu
from jax.experimental.pallas import tpu_sc as plsc

_BATCH = 4096
_IN_DIM = 8192
_OUT_DIM = 16384

_NW = 32
_ROWS_PER_W = _BATCH // _NW
_R = 8
_NBLK = _ROWS_PER_W // _R
_OC = 512
_NCH = _OUT_DIM // _OC
_GPC = _OC // 16
_OCHUNK = _R * _OC

_mesh = plsc.VectorSubcoreMesh(core_axis_name="c", subcore_axis_name="s")


@functools.partial(
    pl.kernel,
    out_type=jax.ShapeDtypeStruct((_BATCH * _OUT_DIM,), jnp.float32),
    mesh=_mesh,
    compiler_params=pltpu.CompilerParams(needs_layout_passes=False),
    scratch_types=[
        pltpu.VMEM((_OUT_DIM,), jnp.int32),
        pltpu.VMEM((_OUT_DIM,), jnp.int32),
        pltpu.VMEM((_OUT_DIM,), jnp.float32),
        pltpu.VMEM((_R * _IN_DIM,), jnp.float32),
        pltpu.VMEM((2 * _OCHUNK,), jnp.float32),
        pltpu.SemaphoreType.DMA,
    ],
)
def _logic_fwd(x_hbm, iaib_hbm, cacb_hbm, cab_hbm, out_hbm,
               iaib_v, cacb_v, cab_v, rows_v, ob_v, out_sem):
    wid = lax.axis_index("s") * 2 + lax.axis_index("c")
    pltpu.sync_copy(iaib_hbm, iaib_v)
    pltpu.sync_copy(cacb_hbm, cacb_v)
    pltpu.sync_copy(cab_hbm, cab_v)

    def out_drain():
        pltpu.make_async_copy(
            ob_v.at[pl.ds(0, _OCHUNK)],
            out_hbm.at[pl.ds(0, _OCHUNK)], out_sem).wait()

    def blk_body(blk, _):
        grp = wid * _NBLK + blk
        pltpu.sync_copy(
            x_hbm.at[pl.ds(grp * (_R * _IN_DIM), _R * _IN_DIM)], rows_v)

        def ch_body(ch, _):
            obuf = lax.rem(ch, 2)
            lin = blk * _NCH + ch

            @pl.when(lin >= 2)
            def _():
                out_drain()

            @plsc.parallel_loop(0, _GPC, 1, unroll=4)
            def g_body(g):
                gbase = ch * _OC + g * 16
                v = iaib_v[pl.ds(gbase, 16)]
                ia = lax.bitwise_and(v, 0xFFFF)
                ib = lax.shift_right_logical(v, 16)
                cc = cacb_v[pl.ds(gbase, 16)]
                ca = plsc.bitcast(lax.shift_left(cc, 16), jnp.float32)
                cb = plsc.bitcast(
                    lax.bitwise_and(cc, jnp.int32(-65536)), jnp.float32)
                cab = cab_v[pl.ds(gbase, 16)]
                obase = obuf * _OCHUNK + (g // 8) * 1024 + (g % 8) * 16
                for r in range(_R):
                    rowslc = rows_v.at[pl.ds(r * _IN_DIM, _IN_DIM)]
                    a = plsc.load_gather(rowslc, [ia])
                    b = plsc.load_gather(rowslc, [ib])
                    ob_v[pl.ds(obase + r * 128, 16)] = (
                        a * (ca + cab * b) + cb * b)

            pltpu.async_copy(
                ob_v.at[pl.ds(obuf * _OCHUNK, _OCHUNK)],
                out_hbm.at[pl.ds(grp * (_R * _OUT_DIM) + ch * _OCHUNK,
                                 _OCHUNK)],
                out_sem)
            return 0

        lax.fori_loop(0, _NCH, ch_body, 0)
        return 0

    lax.fori_loop(0, _NBLK, blk_body, 0)
    out_drain()
    out_drain()


@jax.jit
def kernel(x, weights, idx_a, idx_b):
    w = jax.nn.softmax(weights, axis=-1)
    ca = w[:, 2] + w[:, 3]
    cb = w[:, 4] + w[:, 5]
    cab = w[:, 1] - w[:, 2] - w[:, 4]
    iaib = idx_a | (idx_b << 16)
    ca16 = jax.lax.bitcast_convert_type(
        ca.astype(jnp.bfloat16), jnp.uint16).astype(jnp.int32)
    cb16 = jax.lax.bitcast_convert_type(
        cb.astype(jnp.bfloat16), jnp.uint16).astype(jnp.int32)
    cacb = (cb16 << 16) | ca16
    out = _logic_fwd(x.reshape(-1), iaib, cacb, cab)
    out = out.reshape(_BATCH // 8, _OUT_DIM // 128, 8, 128)
    out = out.transpose(0, 2, 1, 3).reshape(_BATCH, _OUT_DIM)
    return out


# speedup vs baseline: 5.6080x; 1.2594x over previous
import functools

import jax
import jax.numpy as jnp
from jax import lax
from jax.experimental import pallas as pl
from jax.experimental.pallas import tpu as pltpu
from jax.experimental.pallas import tpu_sc as plsc

_BATCH = 4096
_IN_DIM = 8192
_OUT_DIM = 16384

_NW = 32
_ROWS_PER_W = _BATCH // _NW
_R = 8
_NBLK = _ROWS_PER_W // _R
_OC = 512
_NCH = _OUT_DIM // _OC
_GPC = _OC // 16
_OCHUNK = _R * _OC

_mesh = plsc.VectorSubcoreMesh(core_axis_name="c", subcore_axis_name="s")


@functools.partial(
    pl.kernel,
    out_type=jax.ShapeDtypeStruct((_BATCH * _OUT_DIM,), jnp.float32),
    mesh=_mesh,
    compiler_params=pltpu.CompilerParams(needs_layout_passes=False),
    scratch_types=[
        pltpu.VMEM((_OUT_DIM,), jnp.int32),
        pltpu.VMEM((_OUT_DIM,), jnp.int32),
        pltpu.VMEM((_OUT_DIM,), jnp.float32),
        pltpu.VMEM((_R * _IN_DIM,), jnp.float32),
        pltpu.VMEM((2 * _OCHUNK,), jnp.float32),
        pltpu.SemaphoreType.DMA,
    ],
)
def _logic_fwd(x_hbm, iaib_hbm, cacb_hbm, cab_hbm, out_hbm,
               iaib_v, cacb_v, cab_v, rows_v, ob_v, out_sem):
    wid = lax.axis_index("s") * 2 + lax.axis_index("c")
    pltpu.sync_copy(iaib_hbm, iaib_v)
    pltpu.sync_copy(cacb_hbm, cacb_v)
    pltpu.sync_copy(cab_hbm, cab_v)

    def out_drain():
        pltpu.make_async_copy(
            ob_v.at[pl.ds(0, _OCHUNK)],
            out_hbm.at[pl.ds(0, _OCHUNK)], out_sem).wait()

    def blk_body(blk, _):
        grp = wid * _NBLK + blk
        pltpu.sync_copy(
            x_hbm.at[pl.ds(grp * (_R * _IN_DIM), _R * _IN_DIM)], rows_v)

        def ch_body(ch, _):
            obuf = lax.rem(ch, 2)
            lin = blk * _NCH + ch

            @pl.when(lin >= 2)
            def _():
                out_drain()

            @plsc.parallel_loop(0, _GPC, 1, unroll=4)
            def g_body(g):
                gbase = ch * _OC + g * 16
                v = iaib_v[pl.ds(gbase, 16)]
                ia = lax.bitwise_and(v, 0xFFFF)
                ib = lax.shift_right_logical(v, 16)
                cc = cacb_v[pl.ds(gbase, 16)]
                ca = plsc.bitcast(lax.shift_left(cc, 16), jnp.float32)
                cb = plsc.bitcast(
                    lax.bitwise_and(cc, jnp.int32(-65536)), jnp.float32)
                cab = cab_v[pl.ds(gbase, 16)]
                obase = obuf * _OCHUNK + (g // 8) * 1024 + (g % 8) * 16
                for r in range(_R):
                    rowslc = rows_v.at[pl.ds(r * 128, _R * _IN_DIM - 896)]
                    a = plsc.load_gather(rowslc, [ia])
                    b = plsc.load_gather(rowslc, [ib])
                    ob_v[pl.ds(obase + r * 128, 16)] = (
                        a * (ca + cab * b) + cb * b)

            pltpu.async_copy(
                ob_v.at[pl.ds(obuf * _OCHUNK, _OCHUNK)],
                out_hbm.at[pl.ds(grp * (_R * _OUT_DIM) + ch * _OCHUNK,
                                 _OCHUNK)],
                out_sem)
            return 0

        lax.fori_loop(0, _NCH, ch_body, 0)
        return 0

    lax.fori_loop(0, _NBLK, blk_body, 0)
    out_drain()
    out_drain()


@jax.jit
def kernel(x, weights, idx_a, idx_b):
    w = jax.nn.softmax(weights, axis=-1)
    ca = w[:, 2] + w[:, 3]
    cb = w[:, 4] + w[:, 5]
    cab = w[:, 1] - w[:, 2] - w[:, 4]
    ta = ((idx_a >> 7) << 10) | (idx_a & 127)
    tb = ((idx_b >> 7) << 10) | (idx_b & 127)
    iaib = ta | (tb << 16)
    ca16 = jax.lax.bitcast_convert_type(
        ca.astype(jnp.bfloat16), jnp.uint16).astype(jnp.int32)
    cb16 = jax.lax.bitcast_convert_type(
        cb.astype(jnp.bfloat16), jnp.uint16).astype(jnp.int32)
    cacb = (cb16 << 16) | ca16
    x_lin = x.reshape(_BATCH // 8, 8, _IN_DIM // 128, 128)
    x_lin = x_lin.transpose(0, 2, 1, 3).reshape(-1)
    out = _logic_fwd(x_lin, iaib, cacb, cab)
    out = out.reshape(_BATCH // 8, _OUT_DIM // 128, 8, 128)
    out = out.transpose(0, 2, 1, 3).reshape(_BATCH, _OUT_DIM)
    return out
